# Initial kernel scaffold; baseline (speedup 1.0000x reference)
import jax
import jax.numpy as jnp
from jax.experimental import pallas as pl


def kernel(x, edge_index, params):
    raise NotImplementedError("write your pallas kernel here")


# speedup vs baseline: 6.0507x; 6.0507x over previous
import functools

import jax
import jax.numpy as jnp
from jax import lax
from jax.experimental import pallas as pl
from jax.experimental.pallas import tpu as pltpu
from jax.experimental.pallas import tpu_sc as plsc

N = 10000
E = 320000
H = 128
NC = 2
NS = 16
NW = NC * NS
EP = E // NW
C = 80
NCH = EP // C
NPAD = 10240
RPT = NPAD // NS
ZR = 128
F32 = jnp.float32
_PREC = lax.Precision.HIGHEST


def _edge_pass(D, gat, with_deg):
    mesh = plsc.VectorSubcoreMesh(core_axis_name="c", subcore_axis_name="s",
                                  num_cores=NC, num_subcores=NS)
    out_type = [jax.ShapeDtypeStruct((NC, NPAD, D), F32)]
    if gat:
        out_type.append(jax.ShapeDtypeStruct((NC, NPAD), F32))
    if with_deg:
        out_type.append(jax.ShapeDtypeStruct((NC, NPAD), F32))
    scratch = [
        pltpu.VMEM((C,), jnp.int32),
        pltpu.VMEM((C,), jnp.int32),
        pltpu.VMEM((C, D), F32),
        pltpu.VMEM_SHARED((NPAD, D), F32),
        pltpu.SemaphoreType.DMA,
    ]
    if gat:
        scratch += [
            pltpu.VMEM((N,), F32),
            pltpu.VMEM((N,), F32),
            pltpu.VMEM((16,), F32),
            pltpu.VMEM((C,), F32),
            pltpu.VMEM_SHARED((NPAD,), F32),
        ]
    if with_deg:
        scratch += [
            pltpu.VMEM((C,), F32),
            pltpu.VMEM_SHARED((NPAD,), F32),
        ]

    def body(*refs):
        n_in = 5 + (3 if gat else 0)
        n_out = 1 + (1 if gat else 0) + (1 if with_deg else 0)
        ins = refs[:n_in]
        outs = refs[n_in:n_in + n_out]
        scr = refs[n_in + n_out:]
        h_hbm, src_hbm, dst_hbm, zrows, zn_hbm = ins[:5]
        if gat:
            asrc_hbm, adst_hbm, m_hbm = ins[5:8]
        acc_out = outs[0]
        k = 1
        if gat:
            den_out = outs[k]
            k += 1
        if with_deg:
            deg_out = outs[k]
        srcv, dstv, rows, acc_sp, sem = scr[:5]
        k = 5
        if gat:
            asrc_l, adst_l, m_l, exbuf, den_sp = scr[k:k + 5]
            k += 5
        if with_deg:
            ones_l, deg_sp = scr[k:k + 2]

        c = lax.axis_index("c")
        s = lax.axis_index("s")
        wid = s * NC + c
        r0 = s * RPT

        if gat:
            pltpu.sync_copy(asrc_hbm, asrc_l)
            pltpu.sync_copy(adst_hbm, adst_l)
            pltpu.sync_copy(m_hbm, m_l)
        for z in range(RPT // ZR):
            pltpu.sync_copy(zrows, acc_sp.at[pl.ds(r0 + z * ZR, ZR)])
        if gat:
            pltpu.sync_copy(zn_hbm, den_sp.at[pl.ds(s * 640, 640)])
        if with_deg:
            pltpu.sync_copy(zn_hbm, deg_sp.at[pl.ds(s * 640, 640)])
            for z in range(C // 16):
                ones_l[pl.ds(z * 16, 16)] = jnp.full((16,), 1.0, F32)
        plsc.subcore_barrier()

        goff = wid * EP

        def chunk(i, _):
            base = goff + i * C
            pltpu.sync_copy(src_hbm.at[pl.ds(base, C)], srcv)
            pltpu.sync_copy(dst_hbm.at[pl.ds(base, C)], dstv)
            pltpu.async_copy(h_hbm.at[srcv], rows, sem).wait()
            if gat:
                mv = m_l[...]
                for j in range(C // 16):
                    sv = srcv[pl.ds(j * 16, 16)]
                    dv = dstv[pl.ds(j * 16, 16)]
                    a = (plsc.load_gather(asrc_l, [sv])
                         + plsc.load_gather(adst_l, [dv]))
                    a = jnp.where(a > 0, a, a * 0.2) - mv
                    exv = jnp.exp(a)
                    exbuf[pl.ds(j * 16, 16)] = exv
                    ev = lax.iota(jnp.int32, 16) + 16 * j

                    def kbody(kk, _, ev=ev, exv=exv):
                        kv = jnp.full((16,), kk, jnp.int32)
                        vals = plsc.load_gather(rows, [ev, kv])
                        plsc.store_scatter(rows, [ev, kv], vals * exv)
                        return 0

                    lax.fori_loop(0, D, kbody, 0)
                pltpu.sync_copy(exbuf, den_sp.at[dstv], add=True)
            if with_deg:
                pltpu.sync_copy(ones_l, deg_sp.at[dstv], add=True)
            pltpu.sync_copy(rows, acc_sp.at[dstv], add=True)
            return 0

        lax.fori_loop(0, NCH, chunk, 0)
        plsc.subcore_barrier()

        for z in range(RPT // ZR):
            rr = r0 + z * ZR
            pltpu.sync_copy(acc_sp.at[pl.ds(rr, ZR)],
                            acc_out.at[c, pl.ds(rr, ZR)])
        if gat:
            pltpu.sync_copy(den_sp.at[pl.ds(s * 640, 640)],
                            den_out.at[c, pl.ds(s * 640, 640)])
        if with_deg:
            pltpu.sync_copy(deg_sp.at[pl.ds(s * 640, 640)],
                            deg_out.at[c, pl.ds(s * 640, 640)])

    return pl.kernel(body, out_type=tuple(out_type), mesh=mesh,
                     scratch_types=scratch,
                     compiler_params=pltpu.CompilerParams(
                         use_tc_tiling_on_sc=False,
                         needs_layout_passes=False))


_G = 10
_R = N // _G


def _specN(d=1):
    return pl.BlockSpec((_R, d), lambda i: (i, 0))


def _spec2(d=1):
    return pl.BlockSpec((NC, _R, d), lambda i: (0, i, 0))


def _specW(a, b=None):
    if b is None:
        return pl.BlockSpec((a,), lambda i: (0,))
    return pl.BlockSpec((a, b), lambda i: (0, 0))


def _dot(a, b):
    return jnp.dot(a, b, preferred_element_type=F32, precision=_PREC)


def _mm_att(x, w, att_s, att_d):
    def body(x_ref, w_ref, s_ref, d_ref, h_o, as_o, ad_o):
        h = _dot(x_ref[...], w_ref[...])
        h_o[...] = h
        as_o[...] = _dot(h, s_ref[...])
        ad_o[...] = _dot(h, d_ref[...])
    di, do = w.shape
    return pl.pallas_call(
        body, grid=(_G,),
        in_specs=[_specN(di), _specW(di, do), _specW(do, 1), _specW(do, 1)],
        out_specs=[_specN(do), _specN(1), _specN(1)],
        out_shape=[jax.ShapeDtypeStruct((N, do), F32),
                   jax.ShapeDtypeStruct((N, 1), F32),
                   jax.ShapeDtypeStruct((N, 1), F32)],
    )(x, w, att_s, att_d)


def _prep_small(asv, adv):
    def body(as_ref, ad_ref, m_o, exs_o):
        asv_ = as_ref[...]
        adv_ = ad_ref[...]
        m = jnp.max(asv_) + jnp.max(adv_)
        a = asv_ + adv_
        a = jnp.where(a > 0, a, a * 0.2)
        m_o[...] = jnp.full((16,), m, F32)
        exs_o[...] = jnp.exp(a - m)
    return pl.pallas_call(
        body,
        out_shape=(jax.ShapeDtypeStruct((16,), F32),
                   jax.ShapeDtypeStruct((N, 1), F32)),
    )(asv, adv)


def _gat_post(acc, den2, exs, htab, bias, bns, bnb, res, wnext, wnext2=None,
              dinv=None, att=None):
    nxt = [w for w in (wnext, wnext2) if w is not None]

    def body(*refs):
        (acc_r, den_r, exs_r, htab_r, b_r, s_r, t_r), k = refs[:7], 7
        res_r = None
        if res is not None:
            res_r = refs[k]; k += 1
        dinv_r = None
        if dinv is not None:
            dinv_r = refs[k]; k += 1
        n_w = len(nxt) + (2 if att is not None else 0)
        w_rs = refs[k:k + n_w]
        o_rs = refs[k + n_w:]
        num = acc_r[0] + acc_r[1] + exs_r[...] * htab_r[...]
        den = den_r[0] + den_r[1] + exs_r[...]
        o = num / den + b_r[...]
        t = o * s_r[...] + t_r[...]
        if res_r is not None:
            t = t + res_r[...]
        g = jnp.maximum(t, 0.0)
        o_rs[0][...] = g
        sc = dinv_r[...] if dinv_r is not None else None
        hns = []
        for j, w_r in enumerate(w_rs[:len(nxt)]):
            hn = _dot(g, w_r[...])
            hns.append(hn)
            o_rs[1 + j][...] = hn * sc if sc is not None else hn
        if att is not None:
            as_w, ad_w = w_rs[len(nxt)], w_rs[len(nxt) + 1]
            o_rs[1 + len(nxt)][...] = _dot(hns[0], as_w[...])
            o_rs[2 + len(nxt)][...] = _dot(hns[0], ad_w[...])

    in_specs = [_spec2(H), _spec2(), _specN(), _specN(H),
                _specW(H), _specW(H), _specW(H)]
    args = [acc, den2, exs, htab, bias, bns, bnb]
    if res is not None:
        in_specs.append(_specN(H)); args.append(res)
    if dinv is not None:
        in_specs.append(_specN()); args.append(dinv)
    for w in nxt:
        in_specs.append(_specW(w.shape[0], w.shape[1])); args.append(w)
    if att is not None:
        for aw in att:
            in_specs.append(_specW(aw.shape[0], 1)); args.append(aw)
    out_shape = [jax.ShapeDtypeStruct((N, H), F32)]
    out_specs = [_specN(H)]
    for w in nxt:
        out_shape.append(jax.ShapeDtypeStruct((N, w.shape[1]), F32))
        out_specs.append(_specN(w.shape[1]))
    if att is not None:
        for _ in att:
            out_shape.append(jax.ShapeDtypeStruct((N, 1), F32))
            out_specs.append(_specN(1))
    return pl.pallas_call(
        body, grid=(_G,), in_specs=in_specs, out_specs=out_specs,
        out_shape=out_shape,
    )(*args)


def _dinv_of(deg2):
    def body(d_ref, o_ref):
        deg = d_ref[0] + d_ref[1] + 1.0
        o_ref[...] = lax.rsqrt(deg)[:, None]
    return pl.pallas_call(
        body, out_shape=jax.ShapeDtypeStruct((N, 1), F32),
    )(deg2)


def _gcn_post(acc, hp, dinv, bias, bns, bnb, wnext=None, scale_next=False):
    D = hp.shape[1]

    def body(*refs):
        acc_r, hp_r, di_r, b_r, s_r, t_r = refs[:6]
        k = 6
        w_r = None
        if wnext is not None:
            w_r = refs[k]; k += 1
        o_rs = refs[k:]
        di = di_r[...]
        o = di * (acc_r[0] + acc_r[1] + hp_r[...]) + b_r[...]
        g = jnp.maximum(o * s_r[...] + t_r[...], 0.0)
        o_rs[0][...] = g
        if w_r is not None:
            hn = _dot(g, w_r[...])
            o_rs[1][...] = hn * di if scale_next else hn

    in_specs = [_spec2(D), _specN(D), _specN(), _specW(D), _specW(D),
                _specW(D)]
    args = [acc, hp, dinv, bias, bns, bnb]
    out_shape = [jax.ShapeDtypeStruct((N, D), F32)]
    out_specs = [_specN(D)]
    if wnext is not None:
        in_specs.append(_specW(wnext.shape[0], wnext.shape[1]))
        args.append(wnext)
        out_shape.append(jax.ShapeDtypeStruct((N, wnext.shape[1]), F32))
        out_specs.append(_specN(wnext.shape[1]))
    return pl.pallas_call(
        body, grid=(_G,), in_specs=in_specs, out_specs=out_specs,
        out_shape=out_shape,
    )(*args)


def _softmax_small(logit):
    def body(l_ref, o_ref):
        l = l_ref[...]
        l = l - jnp.max(l)
        e = jnp.exp(l)
        o_ref[...] = e / jnp.sum(e)
    return pl.pallas_call(
        body, out_shape=jax.ShapeDtypeStruct((N, 1), F32),
    )(logit)


def _heads_grid(r2, c2, wsm, hp):
    leaves, treedef = jax.tree.flatten(hp)

    def body(r_ref, c_ref, w_ref, *refs):
        p_refs = refs[:len(leaves)]
        o_ref = refs[len(leaves)]
        p = jax.tree.unflatten(treedef, [r[...] for r in p_refs])

        def lin(x, q):
            return _dot(x, q["w"]) + q["b"]

        def ln(x, q):
            mu = jnp.mean(x, axis=-1, keepdims=True)
            var = jnp.mean((x - mu) ** 2, axis=-1, keepdims=True)
            return (x - mu) / jnp.sqrt(var + 1e-5) * q["w"] + q["b"]

        relu = lambda v: jnp.maximum(v, 0.0)
        sig = lambda v: 1.0 / (1.0 + jnp.exp(-v))

        rssi = r_ref[...] * w_ref[...]
        q = p["rssi_pred"]
        rssi = relu(ln(lin(rssi, q[0]), q[1]))
        rssi = relu(ln(lin(rssi, q[2]), q[3]))
        rssi = lin(rssi, q[4])

        cqi = c_ref[...]
        q = p["cqi_ch"]
        ch = sig(ln(lin(relu(ln(lin(cqi, q[0]), q[1])), q[2]), q[3]))
        channel_out = cqi * ch * 1.1
        q = p["cqi_sp"]
        sp = relu(ln(lin(cqi, q[0]), q[1]))
        sp = relu(ln(lin(sp, q[2]), q[3]))
        sp = sig(lin(sp, q[4]))
        spatial_out = cqi * sp * 0.9
        comb = jnp.concatenate([channel_out, spatial_out], axis=-1)
        q = p["cqi_fuse"]
        fused = relu(ln(lin(comb, q[0]), q[1]))
        fused = relu(ln(lin(fused, q[2]), q[3]))
        cqi = fused + cqi
        q = p["cqi_pred"]
        cqi = relu(ln(lin(cqi, q[0]), q[1]))
        cqi = relu(ln(lin(cqi, q[2]), q[3]))
        cqi = lin(cqi, q[4])
        o_ref[...] = jnp.concatenate([rssi, cqi], axis=1)

    D2 = H // 2
    wspecs = []
    for lf in leaves:
        wspecs.append(_specW(*lf.shape) if lf.ndim == 2 else _specW(lf.shape[0]))
    return pl.pallas_call(
        body, grid=(_G,),
        in_specs=[_specN(D2), _specN(D2), _specN(1)] + wspecs,
        out_specs=_specN(2),
        out_shape=jax.ShapeDtypeStruct((N, 2), F32),
    )(r2, c2, wsm, *leaves)


def _bn_fold(p):
    s = p["w"] / jnp.sqrt(p["var"] + 1e-5)
    return s, p["b"] - p["mean"] * s


def kernel(x, edge_index, params):
    p = params
    src = edge_index[0].astype(jnp.int32)
    dst = edge_index[1].astype(jnp.int32)
    z128 = jnp.zeros((ZR, H), F32)
    z64 = jnp.zeros((ZR, H // 2), F32)
    zn = jnp.zeros((640,), F32)

    gat128 = _edge_pass(H, gat=True, with_deg=False)
    gat128d = _edge_pass(H, gat=True, with_deg=True)
    gcn128 = _edge_pass(H, gat=False, with_deg=False)
    gcn64 = _edge_pass(H // 2, gat=False, with_deg=False)

    def att2(q):
        return (q["att_src"].reshape(H, 1), q["att_dst"].reshape(H, 1))

    h1, as1, ad1 = _mm_att(x, p["conv1"]["w"], *att2(p["conv1"]))
    m1, exs1 = _prep_small(as1, ad1)
    acc1, den1, deg = gat128d(h1, src, dst, z128, zn, as1.reshape(N),
                              ad1.reshape(N), m1)
    dinv = _dinv_of(deg[:, :N])
    s1, t1 = _bn_fold(p["bn1"])
    g1, h2, as2, ad2 = _gat_post(acc1, den1[:, :N, None], exs1, h1,
                                 p["conv1"]["b"], s1, t1, None,
                                 p["conv2"]["w"], att=att2(p["conv2"]))

    m2, exs2 = _prep_small(as2, ad2)
    acc2, den2 = gat128(h2, src, dst, z128, zn, as2.reshape(N),
                        ad2.reshape(N), m2)
    s2, t2 = _bn_fold(p["bn2"])
    g2, h3, as3, ad3 = _gat_post(acc2, den2[:, :N, None], exs2, h2,
                                 p["conv2"]["b"], s2, t2, g1,
                                 p["conv3"]["w"], att=att2(p["conv3"]))

    m3, exs3 = _prep_small(as3, ad3)
    acc3, den3 = gat128(h3, src, dst, z128, zn, as3.reshape(N),
                        ad3.reshape(N), m3)
    s3, t3 = _bn_fold(p["bn3"])
    g3, hr1, hc1 = _gat_post(acc3, den3[:, :N, None], exs3, h3,
                             p["conv3"]["b"], s3, t3, g2,
                             p["rssi_conv1"]["w"], p["cqi_conv1"]["w"],
                             dinv=dinv)

    accr1 = gcn128(hr1, src, dst, z128, zn)[0]
    sr1, tr1 = _bn_fold(p["bn_rssi1"])
    r1, hr2 = _gcn_post(accr1, hr1, dinv, p["rssi_conv1"]["b"], sr1, tr1,
                        p["rssi_conv2"]["w"], scale_next=True)
    accr2 = gcn64(hr2, src, dst, z64, zn)[0]
    sr2, tr2 = _bn_fold(p["bn_rssi2"])
    r2, logit = _gcn_post(accr2, hr2, dinv, p["rssi_conv2"]["b"], sr2, tr2,
                          p["rssi_attention"]["w"])

    accc1 = gcn128(hc1, src, dst, z128, zn)[0]
    sc1, tc1 = _bn_fold(p["bn_cqi1"])
    c1, hc2 = _gcn_post(accc1, hc1, dinv, p["cqi_conv1"]["b"], sc1, tc1,
                        p["cqi_conv2"]["w"], scale_next=True)
    accc2 = gcn64(hc2, src, dst, z64, zn)[0]
    sc2, tc2 = _bn_fold(p["bn_cqi2"])
    c2 = _gcn_post(accc2, hc2, dinv, p["cqi_conv2"]["b"], sc2, tc2)[0]

    wsm = _softmax_small(logit)
    hp = {"rssi_pred": p["rssi_pred"], "cqi_ch": p["cqi_ch"],
          "cqi_sp": p["cqi_sp"], "cqi_fuse": p["cqi_fuse"],
          "cqi_pred": p["cqi_pred"]}
    return _heads_grid(r2, c2, wsm, hp)
